# baseline (speedup 1.0000x reference)
import jax
import jax.numpy as jnp
from jax.experimental import pallas as pl
from jax.experimental.pallas import tpu as pltpu

_LORA_ALPHA = 32.0


def kernel(x, gate_W, base_W, base_b, lora_A, lora_B):
    B, S, H = x.shape
    E, R, _ = lora_A.shape
    M = B * S
    ER = E * R
    scaling = _LORA_ALPHA / R
    f32 = jnp.float32
    bf16 = jnp.bfloat16

    xm = x.reshape(M, H)
    bias = base_b.reshape(1, H).astype(f32)

    TM = 512
    grid = (M // TM,)

    def body(x_ref, a_ref, g_ref, w_ref, bias_ref, b3_ref, o_ref,
             wb_ref, agb_ref, bmb_ref):
        @pl.when(pl.program_id(0) == 0)
        def _prep_weights():
            wb_ref[...] = w_ref[...].astype(bf16)
            agb_ref[0:ER, :] = a_ref[...].reshape(ER, H).astype(bf16)
            agb_ref[ER:ER + E, :] = g_ref[...].astype(bf16)
            for e in range(E):
                bmb_ref[e * R:(e + 1) * R, :] = (
                    b3_ref[e].T.astype(bf16))

        xb = x_ref[...].astype(bf16)

        def dot_nt(a, b):
            return jax.lax.dot_general(
                a, b, (((1,), (1,)), ((), ())), preferred_element_type=f32)

        def dot_nn(a, b):
            return jax.lax.dot_general(
                a, b, (((1,), (0,)), ((), ())), preferred_element_type=f32)

        u_all = dot_nt(xb, agb_ref[...])
        u = u_all[:, :ER]
        logits = u_all[:, ER:ER + E]

        ei = jax.lax.broadcasted_iota(jnp.int32, (TM, E), 1)
        m1 = jnp.max(logits, axis=1, keepdims=True)
        i1 = jnp.min(jnp.where(logits == m1, ei, E), axis=1, keepdims=True)
        rest = jnp.where(ei == i1, -jnp.inf, logits)
        m2 = jnp.max(rest, axis=1, keepdims=True)
        i2 = jnp.min(jnp.where(rest == m2, ei, E), axis=1, keepdims=True)
        w1 = 1.0 / (1.0 + jnp.exp(m2 - m1))
        w2 = 1.0 - w1

        lane_e = jax.lax.broadcasted_iota(jnp.int32, (TM, ER), 1) // R
        wf = ((lane_e == i1).astype(f32) * w1 + (lane_e == i2).astype(f32) * w2)

        v = (u * (wf * scaling)).astype(bf16)

        out = dot_nt(xb, wb_ref[...]) + bias_ref[...] + dot_nn(v, bmb_ref[...])
        o_ref[...] = out

    out = pl.pallas_call(
        body,
        grid=grid,
        in_specs=[
            pl.BlockSpec((TM, H), lambda i: (i, 0)),
            pl.BlockSpec((E, R, H), lambda i: (0, 0, 0)),
            pl.BlockSpec((E, H), lambda i: (0, 0)),
            pl.BlockSpec((H, H), lambda i: (0, 0)),
            pl.BlockSpec((1, H), lambda i: (0, 0)),
            pl.BlockSpec((E, H, R), lambda i: (0, 0, 0)),
        ],
        out_specs=pl.BlockSpec((TM, H), lambda i: (i, 0)),
        out_shape=jax.ShapeDtypeStruct((M, H), f32),
        scratch_shapes=[pltpu.VMEM((H, H), bf16),
                        pltpu.VMEM((ER + E, H), bf16),
                        pltpu.VMEM((ER, H), bf16)],
        compiler_params=pltpu.CompilerParams(
            dimension_semantics=("arbitrary",)),
    )(xm, lora_A, gate_W, base_W, bias, lora_B)
    return out.reshape(B, S, H)


# speedup vs baseline: 1.0430x; 1.0430x over previous
import jax
import jax.numpy as jnp
from jax.experimental import pallas as pl
from jax.experimental.pallas import tpu as pltpu

_LORA_ALPHA = 32.0


def kernel(x, gate_W, base_W, base_b, lora_A, lora_B):
    B, S, H = x.shape
    E, R, _ = lora_A.shape
    M = B * S
    ER = E * R
    scaling = _LORA_ALPHA / R
    f32 = jnp.float32
    bf16 = jnp.bfloat16

    xm = x.reshape(M, H)
    ag = jnp.concatenate(
        [lora_A.reshape(ER, H), gate_W], axis=0).astype(bf16)
    b_m = jnp.transpose(lora_B, (0, 2, 1)).reshape(ER, H).astype(bf16)
    bias = base_b.reshape(1, H).astype(f32)

    TM = 512
    grid = (M // TM,)

    def body(x_ref, ag_ref, w_ref, bias_ref, bm_ref, o_ref, wb_ref):
        @pl.when(pl.program_id(0) == 0)
        def _cast_w():
            wb_ref[...] = w_ref[...].astype(bf16)

        xb = x_ref[...].astype(bf16)

        def dot_nt(a, b):
            return jax.lax.dot_general(
                a, b, (((1,), (1,)), ((), ())), preferred_element_type=f32)

        def dot_nn(a, b):
            return jax.lax.dot_general(
                a, b, (((1,), (0,)), ((), ())), preferred_element_type=f32)

        u_all = dot_nt(xb, ag_ref[...])
        u = u_all[:, :ER]
        logits = u_all[:, ER:ER + E]

        ei = jax.lax.broadcasted_iota(jnp.int32, (TM, E), 1)
        m1 = jnp.max(logits, axis=1, keepdims=True)
        i1 = jnp.min(jnp.where(logits == m1, ei, E), axis=1, keepdims=True)
        rest = jnp.where(ei == i1, -jnp.inf, logits)
        m2 = jnp.max(rest, axis=1, keepdims=True)
        i2 = jnp.min(jnp.where(rest == m2, ei, E), axis=1, keepdims=True)
        w1 = 1.0 / (1.0 + jnp.exp(m2 - m1))
        w2 = 1.0 - w1

        lane_e = jax.lax.broadcasted_iota(jnp.int32, (TM, ER), 1) // R
        wf = ((lane_e == i1).astype(f32) * w1 + (lane_e == i2).astype(f32) * w2)

        v = (u * (wf * scaling)).astype(bf16)

        out = dot_nt(xb, wb_ref[...]) + bias_ref[...] + dot_nn(v, bm_ref[...])
        o_ref[...] = out

    out = pl.pallas_call(
        body,
        grid=grid,
        in_specs=[
            pl.BlockSpec((TM, H), lambda i: (i, 0)),
            pl.BlockSpec((ER + E, H), lambda i: (0, 0)),
            pl.BlockSpec((H, H), lambda i: (0, 0)),
            pl.BlockSpec((1, H), lambda i: (0, 0)),
            pl.BlockSpec((ER, H), lambda i: (0, 0)),
        ],
        out_specs=pl.BlockSpec((TM, H), lambda i: (i, 0)),
        out_shape=jax.ShapeDtypeStruct((M, H), f32),
        scratch_shapes=[pltpu.VMEM((H, H), bf16)],
        compiler_params=pltpu.CompilerParams(
            dimension_semantics=("arbitrary",)),
    )(xm, ag, base_W, bias, b_m)
    return out.reshape(B, S, H)
